# Initial kernel scaffold; baseline (speedup 1.0000x reference)
import functools

import jax
import jax.numpy as jnp
from jax import lax
from jax.experimental import pallas as pl
from jax.experimental.pallas import tpu as pltpu
from jax.experimental.pallas import tpu_sc as plsc

N = 10000
E = 320000
F_IN = 128
H = 256
HH = 128
G = 64
C = 10

NC = 2
NS = 16
K = 128

ROWS_PER_TILE = N // NS
WB = 125
NWB = ROWS_PER_TILE // WB


def _sc_agg_body(nchunks, table, srcs, dsts, out, sidx_v, didx_v, rows_v, wb_v,
                 acc, sem):
    c = lax.axis_index("c")
    t = lax.axis_index("s")
    epc = nchunks * K

    def zero_body(i, carry):
        wb_v[pl.ds(i * 16, 16)] = jnp.zeros((16,), jnp.float32)
        return carry
    lax.fori_loop(0, (WB * HH) // 16, zero_body, 0)
    wb2 = wb_v.reshape(WB, HH)
    for w in range(NWB):
        r0 = t * ROWS_PER_TILE + w * WB
        pltpu.sync_copy(wb2, acc.at[pl.ds(r0, WB)])
    plsc.subcore_barrier()

    nj = (nchunks + NS - 1) // NS

    def body(j, carry):
        chunk = j * NS + t

        @pl.when(chunk < nchunks)
        def _():
            off = c * epc + chunk * K
            pltpu.sync_copy(srcs.at[pl.ds(off, K)], sidx_v)
            pltpu.sync_copy(dsts.at[pl.ds(off, K)], didx_v)
            pltpu.async_copy(table.at[sidx_v], rows_v, sem).wait()
            pltpu.sync_copy(rows_v, acc.at[didx_v], add=True)
        return carry

    lax.fori_loop(0, nj, body, 0)
    plsc.subcore_barrier()

    for w in range(NWB):
        r0 = t * ROWS_PER_TILE + w * WB
        pltpu.sync_copy(acc.at[pl.ds(r0, WB)], wb2)
        pltpu.sync_copy(wb2, out.at[pl.ds(c * N + r0, WB)])


def _sc_agg(table, srcs, dsts, nchunks):
    mesh = plsc.VectorSubcoreMesh(core_axis_name="c", subcore_axis_name="s",
                                  num_cores=NC, num_subcores=NS)
    body = functools.partial(_sc_agg_body, nchunks)
    f = pl.kernel(
        body,
        out_type=jax.ShapeDtypeStruct((2 * N, HH), jnp.float32),
        mesh=mesh,
        scratch_types=[
            pltpu.VMEM((K,), jnp.int32),
            pltpu.VMEM((K,), jnp.int32),
            pltpu.VMEM((K, HH), jnp.float32),
            pltpu.VMEM((WB * HH,), jnp.float32),
            pltpu.VMEM_SHARED((N, HH), jnp.float32),
            pltpu.SemaphoreType.DMA,
        ],
    )
    return f(table, srcs, dsts)


R = 1000
NG = N // R


def _mlp_core(a, w1_ref, b1_ref, w2_ref, b2_ref, g_ref, bt_ref):
    s = 1.0 / jnp.sqrt(1.0 + 1e-5)
    h = jnp.maximum(jnp.dot(a, w1_ref[...],
                            preferred_element_type=jnp.float32) + b1_ref[...], 0.0)
    h = jnp.maximum(jnp.dot(h, w2_ref[...],
                            preferred_element_type=jnp.float32) + b2_ref[...], 0.0)
    return h * (s * g_ref[...]) + bt_ref[...]


def _mlp1_body(x_ref, agg_ref, eps_ref, w1_ref, b1_ref, w2_ref, b2_ref,
               g_ref, bt_ref, out_ref):
    a = (1.0 + eps_ref[0, 0]) * x_ref[...] + agg_ref[0] + agg_ref[1]
    h = _mlp_core(a, w1_ref, b1_ref, w2_ref, b2_ref, g_ref, bt_ref)
    out_ref[0] = h[:, :HH]
    out_ref[1] = h[:, HH:]


def _full_spec(shape):
    return pl.BlockSpec(shape, lambda i: (0,) * len(shape))


def _mlp1(x, agg, eps, w1, b1, w2, b2, g, bt):
    return pl.pallas_call(
        _mlp1_body,
        grid=(NG,),
        in_specs=[
            pl.BlockSpec((R, F_IN), lambda i: (i, 0)),
            pl.BlockSpec((2, R, HH), lambda i: (0, i, 0)),
            pl.BlockSpec(memory_space=pltpu.SMEM),
            _full_spec((F_IN, H)),
            _full_spec((H,)),
            _full_spec((H, H)),
            _full_spec((H,)),
            _full_spec((H,)),
            _full_spec((H,)),
        ],
        out_specs=pl.BlockSpec((2, R, HH), lambda i: (0, i, 0)),
        out_shape=jax.ShapeDtypeStruct((2, N, HH), jnp.float32),
    )(x, agg, eps, w1, b1, w2, b2, g, bt)


def _mlp23_body(h_ref, agg_ref, eps_ref, w1_ref, b1_ref, w2_ref, b2_ref,
                g_ref, bt_ref, out_ref):
    hcat = jnp.concatenate([h_ref[0], h_ref[1]], axis=1)
    acat = jnp.concatenate([agg_ref[0], agg_ref[1]], axis=1)
    a = (1.0 + eps_ref[0, 0]) * hcat + acat
    h = _mlp_core(a, w1_ref, b1_ref, w2_ref, b2_ref, g_ref, bt_ref)
    out_ref[0] = h[:, :HH]
    out_ref[1] = h[:, HH:]


def _mlp2(h, agg, eps, w1, b1, w2, b2, g, bt):
    return pl.pallas_call(
        _mlp23_body,
        grid=(NG,),
        in_specs=[
            pl.BlockSpec((2, R, HH), lambda i: (0, i, 0)),
            pl.BlockSpec((2, R, HH), lambda i: (0, i, 0)),
            pl.BlockSpec(memory_space=pltpu.SMEM),
            _full_spec((H, H)),
            _full_spec((H,)),
            _full_spec((H, H)),
            _full_spec((H,)),
            _full_spec((H,)),
            _full_spec((H,)),
        ],
        out_specs=pl.BlockSpec((2, R, HH), lambda i: (0, i, 0)),
        out_shape=jax.ShapeDtypeStruct((2, N, HH), jnp.float32),
    )(h, agg, eps, w1, b1, w2, b2, g, bt)


def _mlp3_pool_head_body(h_ref, agg_ref, batch_ref, eps_ref, w1_ref, b1_ref,
                         w2_ref, b2_ref, g_ref, bt_ref, l1w_ref, l1b_ref,
                         l2w_ref, l2b_ref, out_ref, sums_ref, counts_ref):
    i = pl.program_id(0)

    @pl.when(i == 0)
    def _():
        sums_ref[...] = jnp.zeros_like(sums_ref)
        counts_ref[...] = jnp.zeros_like(counts_ref)

    hcat = jnp.concatenate([h_ref[0], h_ref[1]], axis=1)
    acat = jnp.concatenate([agg_ref[0], agg_ref[1]], axis=1)
    a = (1.0 + eps_ref[0, 0]) * hcat + acat
    h = _mlp_core(a, w1_ref, b1_ref, w2_ref, b2_ref, g_ref, bt_ref)

    b = batch_ref[0, 0, :]
    onehot = (b[:, None] == lax.broadcasted_iota(jnp.int32, (R, G), 1)
              ).astype(jnp.float32)
    sums_ref[...] += lax.dot_general(onehot, h, (((0,), (0,)), ((), ())),
                                     preferred_element_type=jnp.float32)
    counts_ref[...] += jnp.sum(onehot, axis=0)[None, :]

    @pl.when(i == NG - 1)
    def _():
        inv = 1.0 / jnp.maximum(counts_ref[0, :], 1.0)
        pooled = sums_ref[...] * inv[:, None]
        z = jnp.maximum(jnp.dot(pooled, l1w_ref[...],
                                preferred_element_type=jnp.float32) + l1b_ref[...],
                        0.0)
        z = jnp.dot(z, l2w_ref[...],
                    preferred_element_type=jnp.float32) + l2b_ref[...]
        m = jnp.max(z, axis=1, keepdims=True)
        lse = m + jnp.log(jnp.sum(jnp.exp(z - m), axis=1, keepdims=True))
        out_ref[...] = z - lse


def _mlp3_pool_head(h, agg, batch3, eps, w1, b1, w2, b2, g, bt,
                    l1w, l1b, l2w, l2b):
    return pl.pallas_call(
        _mlp3_pool_head_body,
        grid=(NG,),
        in_specs=[
            pl.BlockSpec((2, R, HH), lambda i: (0, i, 0)),
            pl.BlockSpec((2, R, HH), lambda i: (0, i, 0)),
            pl.BlockSpec((1, 1, R), lambda i: (i, 0, 0)),
            pl.BlockSpec(memory_space=pltpu.SMEM),
            _full_spec((H, H)),
            _full_spec((H,)),
            _full_spec((H, H)),
            _full_spec((H,)),
            _full_spec((H,)),
            _full_spec((H,)),
            _full_spec((H, H)),
            _full_spec((H,)),
            _full_spec((H, C)),
            _full_spec((C,)),
        ],
        out_specs=pl.BlockSpec((G, C), lambda i: (0, 0)),
        out_shape=jax.ShapeDtypeStruct((G, C), jnp.float32),
        scratch_shapes=[
            pltpu.VMEM((G, H), jnp.float32),
            pltpu.VMEM((1, G), jnp.float32),
        ],
    )(h, agg, batch3, eps, w1, b1, w2, b2, g, bt, l1w, l1b, l2w, l2b)


def kernel(x, edge_index, batch, eps1, c1_W1, c1_b1, c1_W2, c1_b2, c1_g, c1_bt,
           eps2, c2_W1, c2_b1, c2_W2, c2_b2, c2_g, c2_bt,
           eps3, c3_W1, c3_b1, c3_W2, c3_b2, c3_g, c3_bt,
           lin1_W, lin1_b, lin2_W, lin2_b):
    src = edge_index[0]
    dst = edge_index[1]
    src2 = jnp.concatenate([src, src + N])
    dst2 = jnp.concatenate([dst, dst])
    batch3 = batch.reshape(NG, 1, R)
    e1 = jnp.reshape(eps1, (1, 1))
    e2 = jnp.reshape(eps2, (1, 1))
    e3 = jnp.reshape(eps3, (1, 1))
    agg1 = _sc_agg(x, src, dst, E // (2 * K))
    h1 = _mlp1(x, agg1.reshape(2, N, HH), e1,
               c1_W1, c1_b1, c1_W2, c1_b2, c1_g, c1_bt)
    h1f = h1.reshape(2 * N, HH)
    agg2 = _sc_agg(h1f, src2, dst2, E // K)
    h2 = _mlp2(h1, agg2.reshape(2, N, HH), e2,
               c2_W1, c2_b1, c2_W2, c2_b2, c2_g, c2_bt)
    h2f = h2.reshape(2 * N, HH)
    agg3 = _sc_agg(h2f, src2, dst2, E // K)
    return _mlp3_pool_head(h2, agg3.reshape(2, N, HH), batch3,
                           e3, c3_W1, c3_b1, c3_W2, c3_b2,
                           c3_g, c3_bt, lin1_W, lin1_b, lin2_W, lin2_b)


# speedup vs baseline: 4.5343x; 4.5343x over previous
import functools

import jax
import jax.numpy as jnp
from jax import lax
from jax.experimental import pallas as pl
from jax.experimental.pallas import tpu as pltpu
from jax.experimental.pallas import tpu_sc as plsc

N = 10000
E = 320000
F_IN = 128
H = 256
HH = 128
G = 64
C = 10

NC = 2
NS = 16
K = 128

WB = 80
NWB = N // WB


def _sc_agg_body(nchunks, table, srcs, dsts, out, sidx_v, didx_v, rows_v, wb_v,
                 acc, sem):
    c = lax.axis_index("c")
    t = lax.axis_index("s")
    epc = nchunks * K

    def zero_body(i, carry):
        for kk in range(HH // 16):
            wb_v[i, pl.ds(kk * 16, 16)] = jnp.zeros((16,), jnp.float32)
        return carry
    lax.fori_loop(0, WB, zero_body, 0)
    for w in range(8):
        wc = w * NS + t

        @pl.when(wc < NWB)
        def _():
            pltpu.sync_copy(wb_v, acc.at[pl.ds(wc * WB, WB)])
    plsc.subcore_barrier()

    nj = (nchunks + NS - 1) // NS

    def body(j, carry):
        chunk = j * NS + t

        @pl.when(chunk < nchunks)
        def _():
            off = c * epc + chunk * K
            pltpu.sync_copy(srcs.at[pl.ds(off, K)], sidx_v)
            pltpu.sync_copy(dsts.at[pl.ds(off, K)], didx_v)
            pltpu.async_copy(table.at[sidx_v], rows_v, sem).wait()
            pltpu.sync_copy(rows_v, acc.at[didx_v], add=True)
        return carry

    lax.fori_loop(0, nj, body, 0)
    plsc.subcore_barrier()

    for w in range(8):
        wc = w * NS + t

        @pl.when(wc < NWB)
        def _():
            pltpu.sync_copy(acc.at[pl.ds(wc * WB, WB)], wb_v)
            pltpu.sync_copy(wb_v, out.at[pl.ds(c * N + wc * WB, WB)])


def _sc_agg(table, srcs, dsts, nchunks):
    mesh = plsc.VectorSubcoreMesh(core_axis_name="c", subcore_axis_name="s",
                                  num_cores=NC, num_subcores=NS)
    body = functools.partial(_sc_agg_body, nchunks)
    f = pl.kernel(
        body,
        out_type=jax.ShapeDtypeStruct((2 * N, HH), jnp.float32),
        mesh=mesh,
        scratch_types=[
            pltpu.VMEM((K,), jnp.int32),
            pltpu.VMEM((K,), jnp.int32),
            pltpu.VMEM((K, HH), jnp.float32),
            pltpu.VMEM((WB, HH), jnp.float32),
            pltpu.VMEM_SHARED((N, HH), jnp.float32),
            pltpu.SemaphoreType.DMA,
        ],
    )
    return f(table, srcs, dsts)


R = 1000
NG = N // R


def _mlp_core(a, w1_ref, b1_ref, w2_ref, b2_ref, g_ref, bt_ref):
    s = 1.0 / jnp.sqrt(1.0 + 1e-5)
    h = jnp.maximum(jnp.dot(a, w1_ref[...],
                            preferred_element_type=jnp.float32) + b1_ref[...], 0.0)
    h = jnp.maximum(jnp.dot(h, w2_ref[...],
                            preferred_element_type=jnp.float32) + b2_ref[...], 0.0)
    return h * (s * g_ref[...]) + bt_ref[...]


def _mlp1_body(x_ref, agg_ref, eps_ref, w1_ref, b1_ref, w2_ref, b2_ref,
               g_ref, bt_ref, out_ref):
    a = (1.0 + eps_ref[0, 0]) * x_ref[...] + agg_ref[0] + agg_ref[1]
    h = _mlp_core(a, w1_ref, b1_ref, w2_ref, b2_ref, g_ref, bt_ref)
    out_ref[0] = h[:, :HH]
    out_ref[1] = h[:, HH:]


def _full_spec(shape):
    return pl.BlockSpec(shape, lambda i: (0,) * len(shape))


def _mlp1(x, agg, eps, w1, b1, w2, b2, g, bt):
    return pl.pallas_call(
        _mlp1_body,
        grid=(NG,),
        in_specs=[
            pl.BlockSpec((R, F_IN), lambda i: (i, 0)),
            pl.BlockSpec((2, R, HH), lambda i: (0, i, 0)),
            pl.BlockSpec(memory_space=pltpu.SMEM),
            _full_spec((F_IN, H)),
            _full_spec((H,)),
            _full_spec((H, H)),
            _full_spec((H,)),
            _full_spec((H,)),
            _full_spec((H,)),
        ],
        out_specs=pl.BlockSpec((2, R, HH), lambda i: (0, i, 0)),
        out_shape=jax.ShapeDtypeStruct((2, N, HH), jnp.float32),
    )(x, agg, eps, w1, b1, w2, b2, g, bt)


def _mlp23_body(h_ref, agg_ref, eps_ref, w1_ref, b1_ref, w2_ref, b2_ref,
                g_ref, bt_ref, out_ref):
    hcat = jnp.concatenate([h_ref[0], h_ref[1]], axis=1)
    acat = jnp.concatenate([agg_ref[0], agg_ref[1]], axis=1)
    a = (1.0 + eps_ref[0, 0]) * hcat + acat
    h = _mlp_core(a, w1_ref, b1_ref, w2_ref, b2_ref, g_ref, bt_ref)
    out_ref[0] = h[:, :HH]
    out_ref[1] = h[:, HH:]


def _mlp2(h, agg, eps, w1, b1, w2, b2, g, bt):
    return pl.pallas_call(
        _mlp23_body,
        grid=(NG,),
        in_specs=[
            pl.BlockSpec((2, R, HH), lambda i: (0, i, 0)),
            pl.BlockSpec((2, R, HH), lambda i: (0, i, 0)),
            pl.BlockSpec(memory_space=pltpu.SMEM),
            _full_spec((H, H)),
            _full_spec((H,)),
            _full_spec((H, H)),
            _full_spec((H,)),
            _full_spec((H,)),
            _full_spec((H,)),
        ],
        out_specs=pl.BlockSpec((2, R, HH), lambda i: (0, i, 0)),
        out_shape=jax.ShapeDtypeStruct((2, N, HH), jnp.float32),
    )(h, agg, eps, w1, b1, w2, b2, g, bt)


def _mlp3_pool_head_body(h_ref, agg_ref, batch_ref, eps_ref, w1_ref, b1_ref,
                         w2_ref, b2_ref, g_ref, bt_ref, l1w_ref, l1b_ref,
                         l2w_ref, l2b_ref, out_ref, sums_ref, counts_ref):
    i = pl.program_id(0)

    @pl.when(i == 0)
    def _():
        sums_ref[...] = jnp.zeros_like(sums_ref)
        counts_ref[...] = jnp.zeros_like(counts_ref)

    hcat = jnp.concatenate([h_ref[0], h_ref[1]], axis=1)
    acat = jnp.concatenate([agg_ref[0], agg_ref[1]], axis=1)
    a = (1.0 + eps_ref[0, 0]) * hcat + acat
    h = _mlp_core(a, w1_ref, b1_ref, w2_ref, b2_ref, g_ref, bt_ref)

    b = batch_ref[0, 0, :]
    onehot = (b[:, None] == lax.broadcasted_iota(jnp.int32, (R, G), 1)
              ).astype(jnp.float32)
    sums_ref[...] += lax.dot_general(onehot, h, (((0,), (0,)), ((), ())),
                                     preferred_element_type=jnp.float32)
    counts_ref[...] += jnp.sum(onehot, axis=0)[None, :]

    @pl.when(i == NG - 1)
    def _():
        inv = 1.0 / jnp.maximum(counts_ref[0, :], 1.0)
        pooled = sums_ref[...] * inv[:, None]
        z = jnp.maximum(jnp.dot(pooled, l1w_ref[...],
                                preferred_element_type=jnp.float32) + l1b_ref[...],
                        0.0)
        z = jnp.dot(z, l2w_ref[...],
                    preferred_element_type=jnp.float32) + l2b_ref[...]
        m = jnp.max(z, axis=1, keepdims=True)
        lse = m + jnp.log(jnp.sum(jnp.exp(z - m), axis=1, keepdims=True))
        out_ref[...] = z - lse


def _mlp3_pool_head(h, agg, batch3, eps, w1, b1, w2, b2, g, bt,
                    l1w, l1b, l2w, l2b):
    return pl.pallas_call(
        _mlp3_pool_head_body,
        grid=(NG,),
        in_specs=[
            pl.BlockSpec((2, R, HH), lambda i: (0, i, 0)),
            pl.BlockSpec((2, R, HH), lambda i: (0, i, 0)),
            pl.BlockSpec((1, 1, R), lambda i: (i, 0, 0)),
            pl.BlockSpec(memory_space=pltpu.SMEM),
            _full_spec((H, H)),
            _full_spec((H,)),
            _full_spec((H, H)),
            _full_spec((H,)),
            _full_spec((H,)),
            _full_spec((H,)),
            _full_spec((H, H)),
            _full_spec((H,)),
            _full_spec((H, C)),
            _full_spec((C,)),
        ],
        out_specs=pl.BlockSpec((G, C), lambda i: (0, 0)),
        out_shape=jax.ShapeDtypeStruct((G, C), jnp.float32),
        scratch_shapes=[
            pltpu.VMEM((G, H), jnp.float32),
            pltpu.VMEM((1, G), jnp.float32),
        ],
    )(h, agg, batch3, eps, w1, b1, w2, b2, g, bt, l1w, l1b, l2w, l2b)


def kernel(x, edge_index, batch, eps1, c1_W1, c1_b1, c1_W2, c1_b2, c1_g, c1_bt,
           eps2, c2_W1, c2_b1, c2_W2, c2_b2, c2_g, c2_bt,
           eps3, c3_W1, c3_b1, c3_W2, c3_b2, c3_g, c3_bt,
           lin1_W, lin1_b, lin2_W, lin2_b):
    src = edge_index[0]
    dst = edge_index[1]
    src2 = jnp.concatenate([src, src + N])
    dst2 = jnp.concatenate([dst, dst])
    batch3 = batch.reshape(NG, 1, R)
    e1 = jnp.reshape(eps1, (1, 1))
    e2 = jnp.reshape(eps2, (1, 1))
    e3 = jnp.reshape(eps3, (1, 1))
    agg1 = _sc_agg(x, src, dst, E // (2 * K))
    h1 = _mlp1(x, agg1.reshape(2, N, HH), e1,
               c1_W1, c1_b1, c1_W2, c1_b2, c1_g, c1_bt)
    h1f = h1.reshape(2 * N, HH)
    agg2 = _sc_agg(h1f, src2, dst2, E // K)
    h2 = _mlp2(h1, agg2.reshape(2, N, HH), e2,
               c2_W1, c2_b1, c2_W2, c2_b2, c2_g, c2_bt)
    h2f = h2.reshape(2 * N, HH)
    agg3 = _sc_agg(h2f, src2, dst2, E // K)
    return _mlp3_pool_head(h2, agg3.reshape(2, N, HH), batch3,
                           e3, c3_W1, c3_b1, c3_W2, c3_b2,
                           c3_g, c3_bt, lin1_W, lin1_b, lin2_W, lin2_b)


# speedup vs baseline: 6.2934x; 1.3880x over previous
import functools

import jax
import jax.numpy as jnp
from jax import lax
from jax.experimental import pallas as pl
from jax.experimental.pallas import tpu as pltpu
from jax.experimental.pallas import tpu_sc as plsc

N = 10000
E = 320000
F_IN = 128
H = 256
HH = 128
G = 64
C = 10

NC = 2
NS = 16
K = 128

WB = 80
NWB = N // WB


def _sc_agg_body(nchunks, table, srcs, dsts, out,
                 sidx0, sidx1, didx0, didx1, rows0, rows1, wb_v, acc,
                 isem0, isem1, gsem0, gsem1):
    c = lax.axis_index("c")
    t = lax.axis_index("s")
    epc = nchunks * K

    def zero_body(i, carry):
        for kk in range(HH // 16):
            wb_v[i, pl.ds(kk * 16, 16)] = jnp.zeros((16,), jnp.float32)
        return carry
    lax.fori_loop(0, WB, zero_body, 0)
    for w in range(8):
        wc = w * NS + t

        @pl.when(wc < NWB)
        def _():
            pltpu.sync_copy(wb_v, acc.at[pl.ds(wc * WB, WB)])
    plsc.subcore_barrier()

    sidx = (sidx0, sidx1)
    didx = (didx0, didx1)
    rows = (rows0, rows1)
    isem = (isem0, isem1)
    gsem = (gsem0, gsem1)

    def issue_idx(i, p):
        m = i * NS + t

        @pl.when(m < nchunks)
        def _():
            off = c * epc + m * K
            pltpu.make_async_copy(srcs.at[pl.ds(off, K)], sidx[p], isem[p]).start()
            pltpu.make_async_copy(dsts.at[pl.ds(off, K)], didx[p], isem[p]).start()

    def wait_idx_issue_gather(i, p):
        m = i * NS + t

        @pl.when(m < nchunks)
        def _():
            pltpu.make_async_copy(srcs.at[pl.ds(0, K)], sidx[p], isem[p]).wait()
            pltpu.make_async_copy(dsts.at[pl.ds(0, K)], didx[p], isem[p]).wait()
            pltpu.make_async_copy(table.at[sidx[p]], rows[p], gsem[p]).start()

    def wait_gather_scatter(i, p):
        m = i * NS + t

        @pl.when(m < nchunks)
        def _():
            pltpu.make_async_copy(table.at[sidx[p]], rows[p], gsem[p]).wait()
            pltpu.sync_copy(rows[p], acc.at[didx[p]], add=True)

    def step(i, p):
        wait_gather_scatter(i, p)
        issue_idx(i + 2, p)
        wait_idx_issue_gather(i + 1, 1 - p)

    issue_idx(0, 0)
    issue_idx(1, 1)
    wait_idx_issue_gather(0, 0)

    nj = (((nchunks + NS - 1) // NS) + 1) // 2

    def body(jj, carry):
        step(jj * 2, 0)
        step(jj * 2 + 1, 1)
        return carry

    lax.fori_loop(0, nj, body, 0)
    plsc.subcore_barrier()

    for w in range(8):
        wc = w * NS + t

        @pl.when(wc < NWB)
        def _():
            pltpu.sync_copy(acc.at[pl.ds(wc * WB, WB)], wb_v)
            pltpu.sync_copy(wb_v, out.at[pl.ds(c * N + wc * WB, WB)])


def _sc_agg(table, srcs, dsts, nchunks):
    mesh = plsc.VectorSubcoreMesh(core_axis_name="c", subcore_axis_name="s",
                                  num_cores=NC, num_subcores=NS)
    body = functools.partial(_sc_agg_body, nchunks)
    f = pl.kernel(
        body,
        out_type=jax.ShapeDtypeStruct((2 * N, HH), jnp.float32),
        mesh=mesh,
        scratch_types=[
            pltpu.VMEM((K,), jnp.int32),
            pltpu.VMEM((K,), jnp.int32),
            pltpu.VMEM((K,), jnp.int32),
            pltpu.VMEM((K,), jnp.int32),
            pltpu.VMEM((K, HH), jnp.float32),
            pltpu.VMEM((K, HH), jnp.float32),
            pltpu.VMEM((WB, HH), jnp.float32),
            pltpu.VMEM_SHARED((N, HH), jnp.float32),
            pltpu.SemaphoreType.DMA,
            pltpu.SemaphoreType.DMA,
            pltpu.SemaphoreType.DMA,
            pltpu.SemaphoreType.DMA,
        ],
    )
    return f(table, srcs, dsts)


R = 1000
NG = N // R


def _mlp_core(a, w1_ref, b1_ref, w2_ref, b2_ref, g_ref, bt_ref):
    s = 1.0 / jnp.sqrt(1.0 + 1e-5)
    h = jnp.maximum(jnp.dot(a, w1_ref[...],
                            preferred_element_type=jnp.float32) + b1_ref[...], 0.0)
    h = jnp.maximum(jnp.dot(h, w2_ref[...],
                            preferred_element_type=jnp.float32) + b2_ref[...], 0.0)
    return h * (s * g_ref[...]) + bt_ref[...]


def _mlp1_body(x_ref, agg_ref, eps_ref, w1_ref, b1_ref, w2_ref, b2_ref,
               g_ref, bt_ref, out_ref):
    a = (1.0 + eps_ref[0, 0]) * x_ref[...] + agg_ref[0] + agg_ref[1]
    h = _mlp_core(a, w1_ref, b1_ref, w2_ref, b2_ref, g_ref, bt_ref)
    out_ref[0] = h[:, :HH]
    out_ref[1] = h[:, HH:]


def _full_spec(shape):
    return pl.BlockSpec(shape, lambda i: (0,) * len(shape))


def _mlp1(x, agg, eps, w1, b1, w2, b2, g, bt):
    return pl.pallas_call(
        _mlp1_body,
        grid=(NG,),
        in_specs=[
            pl.BlockSpec((R, F_IN), lambda i: (i, 0)),
            pl.BlockSpec((2, R, HH), lambda i: (0, i, 0)),
            pl.BlockSpec(memory_space=pltpu.SMEM),
            _full_spec((F_IN, H)),
            _full_spec((H,)),
            _full_spec((H, H)),
            _full_spec((H,)),
            _full_spec((H,)),
            _full_spec((H,)),
        ],
        out_specs=pl.BlockSpec((2, R, HH), lambda i: (0, i, 0)),
        out_shape=jax.ShapeDtypeStruct((2, N, HH), jnp.float32),
    )(x, agg, eps, w1, b1, w2, b2, g, bt)


def _mlp23_body(h_ref, agg_ref, eps_ref, w1_ref, b1_ref, w2_ref, b2_ref,
                g_ref, bt_ref, out_ref):
    hcat = jnp.concatenate([h_ref[0], h_ref[1]], axis=1)
    acat = jnp.concatenate([agg_ref[0], agg_ref[1]], axis=1)
    a = (1.0 + eps_ref[0, 0]) * hcat + acat
    h = _mlp_core(a, w1_ref, b1_ref, w2_ref, b2_ref, g_ref, bt_ref)
    out_ref[0] = h[:, :HH]
    out_ref[1] = h[:, HH:]


def _mlp2(h, agg, eps, w1, b1, w2, b2, g, bt):
    return pl.pallas_call(
        _mlp23_body,
        grid=(NG,),
        in_specs=[
            pl.BlockSpec((2, R, HH), lambda i: (0, i, 0)),
            pl.BlockSpec((2, R, HH), lambda i: (0, i, 0)),
            pl.BlockSpec(memory_space=pltpu.SMEM),
            _full_spec((H, H)),
            _full_spec((H,)),
            _full_spec((H, H)),
            _full_spec((H,)),
            _full_spec((H,)),
            _full_spec((H,)),
        ],
        out_specs=pl.BlockSpec((2, R, HH), lambda i: (0, i, 0)),
        out_shape=jax.ShapeDtypeStruct((2, N, HH), jnp.float32),
    )(h, agg, eps, w1, b1, w2, b2, g, bt)


def _mlp3_pool_head_body(h_ref, agg_ref, batch_ref, eps_ref, w1_ref, b1_ref,
                         w2_ref, b2_ref, g_ref, bt_ref, l1w_ref, l1b_ref,
                         l2w_ref, l2b_ref, out_ref, sums_ref, counts_ref):
    i = pl.program_id(0)

    @pl.when(i == 0)
    def _():
        sums_ref[...] = jnp.zeros_like(sums_ref)
        counts_ref[...] = jnp.zeros_like(counts_ref)

    hcat = jnp.concatenate([h_ref[0], h_ref[1]], axis=1)
    acat = jnp.concatenate([agg_ref[0], agg_ref[1]], axis=1)
    a = (1.0 + eps_ref[0, 0]) * hcat + acat
    h = _mlp_core(a, w1_ref, b1_ref, w2_ref, b2_ref, g_ref, bt_ref)

    b = batch_ref[0, 0, :]
    onehot = (b[:, None] == lax.broadcasted_iota(jnp.int32, (R, G), 1)
              ).astype(jnp.float32)
    sums_ref[...] += lax.dot_general(onehot, h, (((0,), (0,)), ((), ())),
                                     preferred_element_type=jnp.float32)
    counts_ref[...] += jnp.sum(onehot, axis=0)[None, :]

    @pl.when(i == NG - 1)
    def _():
        inv = 1.0 / jnp.maximum(counts_ref[0, :], 1.0)
        pooled = sums_ref[...] * inv[:, None]
        z = jnp.maximum(jnp.dot(pooled, l1w_ref[...],
                                preferred_element_type=jnp.float32) + l1b_ref[...],
                        0.0)
        z = jnp.dot(z, l2w_ref[...],
                    preferred_element_type=jnp.float32) + l2b_ref[...]
        m = jnp.max(z, axis=1, keepdims=True)
        lse = m + jnp.log(jnp.sum(jnp.exp(z - m), axis=1, keepdims=True))
        out_ref[...] = z - lse


def _mlp3_pool_head(h, agg, batch3, eps, w1, b1, w2, b2, g, bt,
                    l1w, l1b, l2w, l2b):
    return pl.pallas_call(
        _mlp3_pool_head_body,
        grid=(NG,),
        in_specs=[
            pl.BlockSpec((2, R, HH), lambda i: (0, i, 0)),
            pl.BlockSpec((2, R, HH), lambda i: (0, i, 0)),
            pl.BlockSpec((1, 1, R), lambda i: (i, 0, 0)),
            pl.BlockSpec(memory_space=pltpu.SMEM),
            _full_spec((H, H)),
            _full_spec((H,)),
            _full_spec((H, H)),
            _full_spec((H,)),
            _full_spec((H,)),
            _full_spec((H,)),
            _full_spec((H, H)),
            _full_spec((H,)),
            _full_spec((H, C)),
            _full_spec((C,)),
        ],
        out_specs=pl.BlockSpec((G, C), lambda i: (0, 0)),
        out_shape=jax.ShapeDtypeStruct((G, C), jnp.float32),
        scratch_shapes=[
            pltpu.VMEM((G, H), jnp.float32),
            pltpu.VMEM((1, G), jnp.float32),
        ],
    )(h, agg, batch3, eps, w1, b1, w2, b2, g, bt, l1w, l1b, l2w, l2b)


def kernel(x, edge_index, batch, eps1, c1_W1, c1_b1, c1_W2, c1_b2, c1_g, c1_bt,
           eps2, c2_W1, c2_b1, c2_W2, c2_b2, c2_g, c2_bt,
           eps3, c3_W1, c3_b1, c3_W2, c3_b2, c3_g, c3_bt,
           lin1_W, lin1_b, lin2_W, lin2_b):
    src = edge_index[0]
    dst = edge_index[1]
    src2 = jnp.concatenate([src, src + N])
    dst2 = jnp.concatenate([dst, dst])
    batch3 = batch.reshape(NG, 1, R)
    e1 = jnp.reshape(eps1, (1, 1))
    e2 = jnp.reshape(eps2, (1, 1))
    e3 = jnp.reshape(eps3, (1, 1))
    agg1 = _sc_agg(x, src, dst, E // (2 * K))
    h1 = _mlp1(x, agg1.reshape(2, N, HH), e1,
               c1_W1, c1_b1, c1_W2, c1_b2, c1_g, c1_bt)
    h1f = h1.reshape(2 * N, HH)
    agg2 = _sc_agg(h1f, src2, dst2, E // K)
    h2 = _mlp2(h1, agg2.reshape(2, N, HH), e2,
               c2_W1, c2_b1, c2_W2, c2_b2, c2_g, c2_bt)
    h2f = h2.reshape(2 * N, HH)
    agg3 = _sc_agg(h2f, src2, dst2, E // K)
    return _mlp3_pool_head(h2, agg3.reshape(2, N, HH), batch3,
                           e3, c3_W1, c3_b1, c3_W2, c3_b2,
                           c3_g, c3_bt, lin1_W, lin1_b, lin2_W, lin2_b)


# speedup vs baseline: 8.3975x; 1.3343x over previous
import functools

import jax
import jax.numpy as jnp
from jax import lax
from jax.experimental import pallas as pl
from jax.experimental.pallas import tpu as pltpu
from jax.experimental.pallas import tpu_sc as plsc

N = 10000
E = 320000
F_IN = 128
H = 256
HH = 128
G = 64
C = 10

NC = 2
NS = 16
K = 128

WB = 80
NWB = N // WB


def _sc_agg_body(nchunks, table, srcs, dsts, out,
                 sidx0, sidx1, sidx2, sidx3, didx0, didx1, didx2, didx3,
                 rows0, rows1, rows2, acc,
                 isem0, isem1, isem2, isem3, gsem0, gsem1, gsem2,
                 ssem0, ssem1, ssem2):
    c = lax.axis_index("c")
    t = lax.axis_index("s")
    epc = nchunks * K

    def zero_body(i, carry):
        for kk in range(HH // 16):
            rows0[i, pl.ds(kk * 16, 16)] = jnp.zeros((16,), jnp.float32)
        return carry
    lax.fori_loop(0, WB, zero_body, 0)
    zbuf = rows0.at[pl.ds(0, WB)]
    for w in range(8):
        wc = w * NS + t

        @pl.when(wc < NWB)
        def _():
            pltpu.sync_copy(zbuf, acc.at[pl.ds(wc * WB, WB)])
    plsc.subcore_barrier()

    DR = 3
    DI = 4
    sidx = (sidx0, sidx1, sidx2, sidx3)
    didx = (didx0, didx1, didx2, didx3)
    rows = (rows0, rows1, rows2)
    isem = (isem0, isem1, isem2, isem3)
    gsem = (gsem0, gsem1, gsem2)
    ssem = (ssem0, ssem1, ssem2)

    def issue_idx(i, pi):
        m = i * NS + t

        @pl.when(m < nchunks)
        def _():
            off = c * epc + m * K
            pltpu.make_async_copy(srcs.at[pl.ds(off, K)], sidx[pi], isem[pi]).start()
            pltpu.make_async_copy(dsts.at[pl.ds(off, K)], didx[pi], isem[pi]).start()

    def wait_idx_issue_gather(i, pi, pr):
        m = i * NS + t

        @pl.when(m < nchunks)
        def _():
            pltpu.make_async_copy(srcs.at[pl.ds(0, K)], sidx[pi], isem[pi]).wait()
            pltpu.make_async_copy(dsts.at[pl.ds(0, K)], didx[pi], isem[pi]).wait()
            pltpu.make_async_copy(table.at[sidx[pi]], rows[pr], gsem[pr]).start()

    def wait_gather_start_scatter(i, pi, pr):
        m = i * NS + t

        @pl.when(m < nchunks)
        def _():
            pltpu.make_async_copy(table.at[sidx[pi]], rows[pr], gsem[pr]).wait()
            pltpu.async_copy(rows[pr], acc.at[didx[pi]], ssem[pr], add=True)

    def wait_scatter(i, pi, pr):
        m = i * NS + t

        @pl.when(jnp.logical_and(m >= 0, m < nchunks))
        def _():
            pltpu.make_async_copy(rows[pr], acc.at[didx[pi]], ssem[pr]).wait()

    def step(i, u):
        wait_gather_start_scatter(i, u % DI, u % DR)
        wait_scatter(i - 1, (u - 1) % DI, (u - 1) % DR)
        issue_idx(i + 3, (u + 3) % DI)
        wait_idx_issue_gather(i + 1, (u + 1) % DI, (u + 1) % DR)

    issue_idx(0, 0)
    issue_idx(1, 1)
    issue_idx(2, 2)
    wait_idx_issue_gather(0, 0, 0)

    ni = (nchunks + NS - 1) // NS
    UN = 12
    nj = (ni + 1 + UN - 1) // UN

    def body(jj, carry):
        for u in range(UN):
            step(jj * UN + u, u)
        return carry

    lax.fori_loop(0, nj, body, 0)
    plsc.subcore_barrier()

    for w in range(8):
        wc = w * NS + t

        @pl.when(wc < NWB)
        def _():
            pltpu.sync_copy(acc.at[pl.ds(wc * WB, WB)], zbuf)
            pltpu.sync_copy(zbuf, out.at[pl.ds(c * N + wc * WB, WB)])


def _sc_agg(table, srcs, dsts, nchunks):
    mesh = plsc.VectorSubcoreMesh(core_axis_name="c", subcore_axis_name="s",
                                  num_cores=NC, num_subcores=NS)
    body = functools.partial(_sc_agg_body, nchunks)
    f = pl.kernel(
        body,
        out_type=jax.ShapeDtypeStruct((2 * N, HH), jnp.float32),
        mesh=mesh,
        scratch_types=(
            [pltpu.VMEM((K,), jnp.int32)] * 8
            + [pltpu.VMEM((K, HH), jnp.float32)] * 3
            + [pltpu.VMEM_SHARED((N, HH), jnp.float32)]
            + [pltpu.SemaphoreType.DMA] * 10
        ),
    )
    return f(table, srcs, dsts)


R = 1000
NG = N // R


def _mlp_core(a, w1_ref, b1_ref, w2_ref, b2_ref, g_ref, bt_ref):
    s = 1.0 / jnp.sqrt(1.0 + 1e-5)
    h = jnp.maximum(jnp.dot(a, w1_ref[...],
                            preferred_element_type=jnp.float32) + b1_ref[...], 0.0)
    h = jnp.maximum(jnp.dot(h, w2_ref[...],
                            preferred_element_type=jnp.float32) + b2_ref[...], 0.0)
    return h * (s * g_ref[...]) + bt_ref[...]


def _mlp1_body(x_ref, agg_ref, eps_ref, w1_ref, b1_ref, w2_ref, b2_ref,
               g_ref, bt_ref, out_ref):
    a = (1.0 + eps_ref[0, 0]) * x_ref[...] + agg_ref[0] + agg_ref[1]
    h = _mlp_core(a, w1_ref, b1_ref, w2_ref, b2_ref, g_ref, bt_ref)
    out_ref[0] = h[:, :HH]
    out_ref[1] = h[:, HH:]


def _full_spec(shape):
    return pl.BlockSpec(shape, lambda i: (0,) * len(shape))


def _mlp1(x, agg, eps, w1, b1, w2, b2, g, bt):
    return pl.pallas_call(
        _mlp1_body,
        grid=(NG,),
        in_specs=[
            pl.BlockSpec((R, F_IN), lambda i: (i, 0)),
            pl.BlockSpec((2, R, HH), lambda i: (0, i, 0)),
            pl.BlockSpec(memory_space=pltpu.SMEM),
            _full_spec((F_IN, H)),
            _full_spec((H,)),
            _full_spec((H, H)),
            _full_spec((H,)),
            _full_spec((H,)),
            _full_spec((H,)),
        ],
        out_specs=pl.BlockSpec((2, R, HH), lambda i: (0, i, 0)),
        out_shape=jax.ShapeDtypeStruct((2, N, HH), jnp.float32),
    )(x, agg, eps, w1, b1, w2, b2, g, bt)


def _mlp23_body(h_ref, agg_ref, eps_ref, w1_ref, b1_ref, w2_ref, b2_ref,
                g_ref, bt_ref, out_ref):
    hcat = jnp.concatenate([h_ref[0], h_ref[1]], axis=1)
    acat = jnp.concatenate([agg_ref[0], agg_ref[1]], axis=1)
    a = (1.0 + eps_ref[0, 0]) * hcat + acat
    h = _mlp_core(a, w1_ref, b1_ref, w2_ref, b2_ref, g_ref, bt_ref)
    out_ref[0] = h[:, :HH]
    out_ref[1] = h[:, HH:]


def _mlp2(h, agg, eps, w1, b1, w2, b2, g, bt):
    return pl.pallas_call(
        _mlp23_body,
        grid=(NG,),
        in_specs=[
            pl.BlockSpec((2, R, HH), lambda i: (0, i, 0)),
            pl.BlockSpec((2, R, HH), lambda i: (0, i, 0)),
            pl.BlockSpec(memory_space=pltpu.SMEM),
            _full_spec((H, H)),
            _full_spec((H,)),
            _full_spec((H, H)),
            _full_spec((H,)),
            _full_spec((H,)),
            _full_spec((H,)),
        ],
        out_specs=pl.BlockSpec((2, R, HH), lambda i: (0, i, 0)),
        out_shape=jax.ShapeDtypeStruct((2, N, HH), jnp.float32),
    )(h, agg, eps, w1, b1, w2, b2, g, bt)


def _mlp3_pool_head_body(h_ref, agg_ref, batch_ref, eps_ref, w1_ref, b1_ref,
                         w2_ref, b2_ref, g_ref, bt_ref, l1w_ref, l1b_ref,
                         l2w_ref, l2b_ref, out_ref, sums_ref, counts_ref):
    i = pl.program_id(0)

    @pl.when(i == 0)
    def _():
        sums_ref[...] = jnp.zeros_like(sums_ref)
        counts_ref[...] = jnp.zeros_like(counts_ref)

    hcat = jnp.concatenate([h_ref[0], h_ref[1]], axis=1)
    acat = jnp.concatenate([agg_ref[0], agg_ref[1]], axis=1)
    a = (1.0 + eps_ref[0, 0]) * hcat + acat
    h = _mlp_core(a, w1_ref, b1_ref, w2_ref, b2_ref, g_ref, bt_ref)

    b = batch_ref[0, 0, :]
    onehot = (b[:, None] == lax.broadcasted_iota(jnp.int32, (R, G), 1)
              ).astype(jnp.float32)
    sums_ref[...] += lax.dot_general(onehot, h, (((0,), (0,)), ((), ())),
                                     preferred_element_type=jnp.float32)
    counts_ref[...] += jnp.sum(onehot, axis=0)[None, :]

    @pl.when(i == NG - 1)
    def _():
        inv = 1.0 / jnp.maximum(counts_ref[0, :], 1.0)
        pooled = sums_ref[...] * inv[:, None]
        z = jnp.maximum(jnp.dot(pooled, l1w_ref[...],
                                preferred_element_type=jnp.float32) + l1b_ref[...],
                        0.0)
        z = jnp.dot(z, l2w_ref[...],
                    preferred_element_type=jnp.float32) + l2b_ref[...]
        m = jnp.max(z, axis=1, keepdims=True)
        lse = m + jnp.log(jnp.sum(jnp.exp(z - m), axis=1, keepdims=True))
        out_ref[...] = z - lse


def _mlp3_pool_head(h, agg, batch3, eps, w1, b1, w2, b2, g, bt,
                    l1w, l1b, l2w, l2b):
    return pl.pallas_call(
        _mlp3_pool_head_body,
        grid=(NG,),
        in_specs=[
            pl.BlockSpec((2, R, HH), lambda i: (0, i, 0)),
            pl.BlockSpec((2, R, HH), lambda i: (0, i, 0)),
            pl.BlockSpec((1, 1, R), lambda i: (i, 0, 0)),
            pl.BlockSpec(memory_space=pltpu.SMEM),
            _full_spec((H, H)),
            _full_spec((H,)),
            _full_spec((H, H)),
            _full_spec((H,)),
            _full_spec((H,)),
            _full_spec((H,)),
            _full_spec((H, H)),
            _full_spec((H,)),
            _full_spec((H, C)),
            _full_spec((C,)),
        ],
        out_specs=pl.BlockSpec((G, C), lambda i: (0, 0)),
        out_shape=jax.ShapeDtypeStruct((G, C), jnp.float32),
        scratch_shapes=[
            pltpu.VMEM((G, H), jnp.float32),
            pltpu.VMEM((1, G), jnp.float32),
        ],
    )(h, agg, batch3, eps, w1, b1, w2, b2, g, bt, l1w, l1b, l2w, l2b)


def kernel(x, edge_index, batch, eps1, c1_W1, c1_b1, c1_W2, c1_b2, c1_g, c1_bt,
           eps2, c2_W1, c2_b1, c2_W2, c2_b2, c2_g, c2_bt,
           eps3, c3_W1, c3_b1, c3_W2, c3_b2, c3_g, c3_bt,
           lin1_W, lin1_b, lin2_W, lin2_b):
    src = edge_index[0]
    dst = edge_index[1]
    src2 = jnp.concatenate([src, src + N])
    dst2 = jnp.concatenate([dst, dst])
    batch3 = batch.reshape(NG, 1, R)
    e1 = jnp.reshape(eps1, (1, 1))
    e2 = jnp.reshape(eps2, (1, 1))
    e3 = jnp.reshape(eps3, (1, 1))
    agg1 = _sc_agg(x, src, dst, E // (2 * K))
    h1 = _mlp1(x, agg1.reshape(2, N, HH), e1,
               c1_W1, c1_b1, c1_W2, c1_b2, c1_g, c1_bt)
    h1f = h1.reshape(2 * N, HH)
    agg2 = _sc_agg(h1f, src2, dst2, E // K)
    h2 = _mlp2(h1, agg2.reshape(2, N, HH), e2,
               c2_W1, c2_b1, c2_W2, c2_b2, c2_g, c2_bt)
    h2f = h2.reshape(2 * N, HH)
    agg3 = _sc_agg(h2f, src2, dst2, E // K)
    return _mlp3_pool_head(h2, agg3.reshape(2, N, HH), batch3,
                           e3, c3_W1, c3_b1, c3_W2, c3_b2,
                           c3_g, c3_bt, lin1_W, lin1_b, lin2_W, lin2_b)


# speedup vs baseline: 8.4144x; 1.0020x over previous
import functools

import jax
import jax.numpy as jnp
from jax import lax
from jax.experimental import pallas as pl
from jax.experimental.pallas import tpu as pltpu
from jax.experimental.pallas import tpu_sc as plsc

N = 10000
E = 320000
F_IN = 128
H = 256
HH = 128
G = 64
C = 10

NC = 2
NS = 16
K = 128

WB = 80
NWB = N // WB


def _sc_agg_body(nchunks, edge_split, table, srcs, dsts, out,
                 sidx0, sidx1, sidx2, sidx3, didx0, didx1, didx2, didx3,
                 rows0, rows1, rows2, acc,
                 isem0, isem1, isem2, isem3, gsem0, gsem1, gsem2,
                 ssem0, ssem1, ssem2):
    c = lax.axis_index("c")
    t = lax.axis_index("s")
    eoff = c * (nchunks * K) if edge_split else 0
    shift = 0 if edge_split else c * N

    def zero_body(i, carry):
        for kk in range(HH // 16):
            rows0[i, pl.ds(kk * 16, 16)] = jnp.zeros((16,), jnp.float32)
        return carry
    lax.fori_loop(0, WB, zero_body, 0)
    zbuf = rows0.at[pl.ds(0, WB)]
    for w in range(8):
        wc = w * NS + t

        @pl.when(wc < NWB)
        def _():
            pltpu.sync_copy(zbuf, acc.at[pl.ds(wc * WB, WB)])
    plsc.subcore_barrier()

    DR = 3
    DI = 4
    sidx = (sidx0, sidx1, sidx2, sidx3)
    didx = (didx0, didx1, didx2, didx3)
    rows = (rows0, rows1, rows2)
    isem = (isem0, isem1, isem2, isem3)
    gsem = (gsem0, gsem1, gsem2)
    ssem = (ssem0, ssem1, ssem2)

    def issue_idx(i, pi):
        m = i * NS + t

        @pl.when(m < nchunks)
        def _():
            off = eoff + m * K
            pltpu.make_async_copy(srcs.at[pl.ds(off, K)], sidx[pi], isem[pi]).start()
            pltpu.make_async_copy(dsts.at[pl.ds(off, K)], didx[pi], isem[pi]).start()

    def wait_idx_issue_gather(i, pi, pr):
        m = i * NS + t

        @pl.when(m < nchunks)
        def _():
            pltpu.make_async_copy(srcs.at[pl.ds(0, K)], sidx[pi], isem[pi]).wait()
            pltpu.make_async_copy(dsts.at[pl.ds(0, K)], didx[pi], isem[pi]).wait()
            if not edge_split:
                for kk in range(K // 16):
                    sl = pl.ds(kk * 16, 16)
                    sidx[pi][sl] = sidx[pi][sl] + shift
            pltpu.make_async_copy(table.at[sidx[pi]], rows[pr], gsem[pr]).start()

    def wait_gather_start_scatter(i, pi, pr):
        m = i * NS + t

        @pl.when(m < nchunks)
        def _():
            pltpu.make_async_copy(table.at[sidx[pi]], rows[pr], gsem[pr]).wait()
            pltpu.async_copy(rows[pr], acc.at[didx[pi]], ssem[pr], add=True)

    def wait_scatter(i, pi, pr):
        m = i * NS + t

        @pl.when(jnp.logical_and(m >= 0, m < nchunks))
        def _():
            pltpu.make_async_copy(rows[pr], acc.at[didx[pi]], ssem[pr]).wait()

    def step(i, u):
        wait_gather_start_scatter(i, u % DI, u % DR)
        wait_scatter(i - 1, (u - 1) % DI, (u - 1) % DR)
        issue_idx(i + 3, (u + 3) % DI)
        wait_idx_issue_gather(i + 1, (u + 1) % DI, (u + 1) % DR)

    issue_idx(0, 0)
    issue_idx(1, 1)
    issue_idx(2, 2)
    wait_idx_issue_gather(0, 0, 0)

    ni = (nchunks + NS - 1) // NS
    UN = 12
    nj = (ni + 1 + UN - 1) // UN

    def body(jj, carry):
        for u in range(UN):
            step(jj * UN + u, u)
        return carry

    lax.fori_loop(0, nj, body, 0)
    plsc.subcore_barrier()

    for w in range(8):
        wc = w * NS + t

        @pl.when(wc < NWB)
        def _():
            pltpu.sync_copy(acc.at[pl.ds(wc * WB, WB)],
                            out.at[pl.ds(c * N + wc * WB, WB)])


def _sc_agg(table, srcs, dsts, nchunks, edge_split):
    mesh = plsc.VectorSubcoreMesh(core_axis_name="c", subcore_axis_name="s",
                                  num_cores=NC, num_subcores=NS)
    body = functools.partial(_sc_agg_body, nchunks, edge_split)
    f = pl.kernel(
        body,
        out_type=jax.ShapeDtypeStruct((2 * N, HH), jnp.float32),
        mesh=mesh,
        scratch_types=(
            [pltpu.VMEM((K,), jnp.int32)] * 8
            + [pltpu.VMEM((K, HH), jnp.float32)] * 3
            + [pltpu.VMEM_SHARED((N, HH), jnp.float32)]
            + [pltpu.SemaphoreType.DMA] * 10
        ),
    )
    return f(table, srcs, dsts)


R = 1000
NG = N // R


def _mlp_core(a, w1_ref, b1_ref, w2_ref, b2_ref, g_ref, bt_ref):
    s = 1.0 / jnp.sqrt(1.0 + 1e-5)
    h = jnp.maximum(jnp.dot(a, w1_ref[...],
                            preferred_element_type=jnp.float32) + b1_ref[...], 0.0)
    h = jnp.maximum(jnp.dot(h, w2_ref[...],
                            preferred_element_type=jnp.float32) + b2_ref[...], 0.0)
    return h * (s * g_ref[...]) + bt_ref[...]


def _mlp1_body(x_ref, agg_ref, eps_ref, w1_ref, b1_ref, w2_ref, b2_ref,
               g_ref, bt_ref, out_ref):
    a = (1.0 + eps_ref[0, 0]) * x_ref[...] + agg_ref[0] + agg_ref[1]
    h = _mlp_core(a, w1_ref, b1_ref, w2_ref, b2_ref, g_ref, bt_ref)
    out_ref[0] = h[:, :HH]
    out_ref[1] = h[:, HH:]


def _full_spec(shape):
    return pl.BlockSpec(shape, lambda i: (0,) * len(shape))


def _mlp1(x, agg, eps, w1, b1, w2, b2, g, bt):
    return pl.pallas_call(
        _mlp1_body,
        grid=(NG,),
        in_specs=[
            pl.BlockSpec((R, F_IN), lambda i: (i, 0)),
            pl.BlockSpec((2, R, HH), lambda i: (0, i, 0)),
            pl.BlockSpec(memory_space=pltpu.SMEM),
            _full_spec((F_IN, H)),
            _full_spec((H,)),
            _full_spec((H, H)),
            _full_spec((H,)),
            _full_spec((H,)),
            _full_spec((H,)),
        ],
        out_specs=pl.BlockSpec((2, R, HH), lambda i: (0, i, 0)),
        out_shape=jax.ShapeDtypeStruct((2, N, HH), jnp.float32),
    )(x, agg, eps, w1, b1, w2, b2, g, bt)


def _mlp23_body(h_ref, agg_ref, eps_ref, w1_ref, b1_ref, w2_ref, b2_ref,
                g_ref, bt_ref, out_ref):
    hcat = jnp.concatenate([h_ref[0], h_ref[1]], axis=1)
    acat = jnp.concatenate([agg_ref[0], agg_ref[1]], axis=1)
    a = (1.0 + eps_ref[0, 0]) * hcat + acat
    h = _mlp_core(a, w1_ref, b1_ref, w2_ref, b2_ref, g_ref, bt_ref)
    out_ref[0] = h[:, :HH]
    out_ref[1] = h[:, HH:]


def _mlp2(h, agg, eps, w1, b1, w2, b2, g, bt):
    return pl.pallas_call(
        _mlp23_body,
        grid=(NG,),
        in_specs=[
            pl.BlockSpec((2, R, HH), lambda i: (0, i, 0)),
            pl.BlockSpec((2, R, HH), lambda i: (0, i, 0)),
            pl.BlockSpec(memory_space=pltpu.SMEM),
            _full_spec((H, H)),
            _full_spec((H,)),
            _full_spec((H, H)),
            _full_spec((H,)),
            _full_spec((H,)),
            _full_spec((H,)),
        ],
        out_specs=pl.BlockSpec((2, R, HH), lambda i: (0, i, 0)),
        out_shape=jax.ShapeDtypeStruct((2, N, HH), jnp.float32),
    )(h, agg, eps, w1, b1, w2, b2, g, bt)


def _mlp3_pool_head_body(h_ref, agg_ref, batch_ref, eps_ref, w1_ref, b1_ref,
                         w2_ref, b2_ref, g_ref, bt_ref, l1w_ref, l1b_ref,
                         l2w_ref, l2b_ref, out_ref, sums_ref, counts_ref):
    i = pl.program_id(0)

    @pl.when(i == 0)
    def _():
        sums_ref[...] = jnp.zeros_like(sums_ref)
        counts_ref[...] = jnp.zeros_like(counts_ref)

    hcat = jnp.concatenate([h_ref[0], h_ref[1]], axis=1)
    acat = jnp.concatenate([agg_ref[0], agg_ref[1]], axis=1)
    a = (1.0 + eps_ref[0, 0]) * hcat + acat
    h = _mlp_core(a, w1_ref, b1_ref, w2_ref, b2_ref, g_ref, bt_ref)

    b = batch_ref[0, 0, :]
    onehot = (b[:, None] == lax.broadcasted_iota(jnp.int32, (R, G), 1)
              ).astype(jnp.float32)
    sums_ref[...] += lax.dot_general(onehot, h, (((0,), (0,)), ((), ())),
                                     preferred_element_type=jnp.float32)
    counts_ref[...] += jnp.sum(onehot, axis=0)[None, :]

    @pl.when(i == NG - 1)
    def _():
        inv = 1.0 / jnp.maximum(counts_ref[0, :], 1.0)
        pooled = sums_ref[...] * inv[:, None]
        z = jnp.maximum(jnp.dot(pooled, l1w_ref[...],
                                preferred_element_type=jnp.float32) + l1b_ref[...],
                        0.0)
        z = jnp.dot(z, l2w_ref[...],
                    preferred_element_type=jnp.float32) + l2b_ref[...]
        m = jnp.max(z, axis=1, keepdims=True)
        lse = m + jnp.log(jnp.sum(jnp.exp(z - m), axis=1, keepdims=True))
        out_ref[...] = z - lse


def _mlp3_pool_head(h, agg, batch3, eps, w1, b1, w2, b2, g, bt,
                    l1w, l1b, l2w, l2b):
    return pl.pallas_call(
        _mlp3_pool_head_body,
        grid=(NG,),
        in_specs=[
            pl.BlockSpec((2, R, HH), lambda i: (0, i, 0)),
            pl.BlockSpec((2, R, HH), lambda i: (0, i, 0)),
            pl.BlockSpec((1, 1, R), lambda i: (i, 0, 0)),
            pl.BlockSpec(memory_space=pltpu.SMEM),
            _full_spec((H, H)),
            _full_spec((H,)),
            _full_spec((H, H)),
            _full_spec((H,)),
            _full_spec((H,)),
            _full_spec((H,)),
            _full_spec((H, H)),
            _full_spec((H,)),
            _full_spec((H, C)),
            _full_spec((C,)),
        ],
        out_specs=pl.BlockSpec((G, C), lambda i: (0, 0)),
        out_shape=jax.ShapeDtypeStruct((G, C), jnp.float32),
        scratch_shapes=[
            pltpu.VMEM((G, H), jnp.float32),
            pltpu.VMEM((1, G), jnp.float32),
        ],
    )(h, agg, batch3, eps, w1, b1, w2, b2, g, bt, l1w, l1b, l2w, l2b)


def kernel(x, edge_index, batch, eps1, c1_W1, c1_b1, c1_W2, c1_b2, c1_g, c1_bt,
           eps2, c2_W1, c2_b1, c2_W2, c2_b2, c2_g, c2_bt,
           eps3, c3_W1, c3_b1, c3_W2, c3_b2, c3_g, c3_bt,
           lin1_W, lin1_b, lin2_W, lin2_b):
    src = edge_index[0]
    dst = edge_index[1]
    batch3 = batch.reshape(NG, 1, R)
    e1 = jnp.reshape(eps1, (1, 1))
    e2 = jnp.reshape(eps2, (1, 1))
    e3 = jnp.reshape(eps3, (1, 1))
    agg1 = _sc_agg(x, src, dst, E // (2 * K), edge_split=True)
    h1 = _mlp1(x, agg1.reshape(2, N, HH), e1,
               c1_W1, c1_b1, c1_W2, c1_b2, c1_g, c1_bt)
    h1f = h1.reshape(2 * N, HH)
    agg2 = _sc_agg(h1f, src, dst, E // K, edge_split=False)
    h2 = _mlp2(h1, agg2.reshape(2, N, HH), e2,
               c2_W1, c2_b1, c2_W2, c2_b2, c2_g, c2_bt)
    h2f = h2.reshape(2 * N, HH)
    agg3 = _sc_agg(h2f, src, dst, E // K, edge_split=False)
    return _mlp3_pool_head(h2, agg3.reshape(2, N, HH), batch3,
                           e3, c3_W1, c3_b1, c3_W2, c3_b2,
                           c3_g, c3_bt, lin1_W, lin1_b, lin2_W, lin2_b)


# speedup vs baseline: 8.4490x; 1.0041x over previous
import functools

import jax
import jax.numpy as jnp
from jax import lax
from jax.experimental import pallas as pl
from jax.experimental.pallas import tpu as pltpu
from jax.experimental.pallas import tpu_sc as plsc

N = 10000
E = 320000
F_IN = 128
H = 256
HH = 128
G = 64
C = 10

NC = 2
NS = 16
K = 128

WB = 80
NWB = N // WB


def _sc_agg_body(nchunks, edge_split, table, srcs, dsts, out,
                 sidx0, sidx1, sidx2, sidx3, didx0, didx1, didx2, didx3,
                 rows0, rows1, rows2, acc,
                 isem0, isem1, isem2, isem3, gsem0, gsem1, gsem2,
                 ssem0, ssem1, ssem2):
    c = lax.axis_index("c")
    t = lax.axis_index("s")
    eoff = c * (nchunks * K) if edge_split else 0
    shift = 0 if edge_split else c * N

    def zero_body(i, carry):
        for kk in range(HH // 16):
            rows0[i, pl.ds(kk * 16, 16)] = jnp.zeros((16,), jnp.float32)
        return carry
    lax.fori_loop(0, WB, zero_body, 0)
    zbuf = rows0.at[pl.ds(0, WB)]
    for w in range(8):
        wc = w * NS + t

        @pl.when(wc < NWB)
        def _():
            pltpu.sync_copy(zbuf, acc.at[pl.ds(wc * WB, WB)])
    plsc.subcore_barrier()

    DR = 3
    DI = 4
    sidx = (sidx0, sidx1, sidx2, sidx3)
    didx = (didx0, didx1, didx2, didx3)
    rows = (rows0, rows1, rows2)
    isem = (isem0, isem1, isem2, isem3)
    gsem = (gsem0, gsem1, gsem2)
    ssem = (ssem0, ssem1, ssem2)

    def issue_idx(i, pi):
        m = i * NS + t

        @pl.when(m < nchunks)
        def _():
            off = eoff + m * K
            pltpu.make_async_copy(srcs.at[pl.ds(off, K)], sidx[pi], isem[pi]).start()
            pltpu.make_async_copy(dsts.at[pl.ds(off, K)], didx[pi], isem[pi]).start()

    def wait_idx_issue_gather(i, pi, pr):
        m = i * NS + t

        @pl.when(m < nchunks)
        def _():
            pltpu.make_async_copy(srcs.at[pl.ds(0, K)], sidx[pi], isem[pi]).wait()
            pltpu.make_async_copy(dsts.at[pl.ds(0, K)], didx[pi], isem[pi]).wait()
            if not edge_split:
                for kk in range(K // 16):
                    sl = pl.ds(kk * 16, 16)
                    sidx[pi][sl] = sidx[pi][sl] + shift
            pltpu.make_async_copy(table.at[sidx[pi]], rows[pr], gsem[pr]).start()

    def wait_gather_start_scatter(i, pi, pr):
        m = i * NS + t

        @pl.when(m < nchunks)
        def _():
            pltpu.make_async_copy(table.at[sidx[pi]], rows[pr], gsem[pr]).wait()
            pltpu.async_copy(rows[pr], acc.at[didx[pi]], ssem[pr], add=True)

    def wait_scatter(i, pi, pr):
        m = i * NS + t

        @pl.when(jnp.logical_and(m >= 0, m < nchunks))
        def _():
            pltpu.make_async_copy(rows[pr], acc.at[didx[pi]], ssem[pr]).wait()

    def step(i, u):
        wait_gather_start_scatter(i, u % DI, u % DR)
        wait_scatter(i - 1, (u - 1) % DI, (u - 1) % DR)
        issue_idx(i + 3, (u + 3) % DI)
        wait_idx_issue_gather(i + 1, (u + 1) % DI, (u + 1) % DR)

    issue_idx(0, 0)
    issue_idx(1, 1)
    issue_idx(2, 2)
    wait_idx_issue_gather(0, 0, 0)

    ni = (nchunks + NS - 1) // NS
    UN = 12
    nj = (ni + 1 + UN - 1) // UN

    def body(jj, carry):
        for u in range(UN):
            step(jj * UN + u, u)
        return carry

    lax.fori_loop(0, nj, body, 0)
    plsc.subcore_barrier()

    for w in range(8):
        wc = w * NS + t

        @pl.when(wc < NWB)
        def _():
            pltpu.sync_copy(acc.at[pl.ds(wc * WB, WB)],
                            out.at[pl.ds(c * N + wc * WB, WB)])


def _sc_agg(table, srcs, dsts, nchunks, edge_split):
    mesh = plsc.VectorSubcoreMesh(core_axis_name="c", subcore_axis_name="s",
                                  num_cores=NC, num_subcores=NS)
    body = functools.partial(_sc_agg_body, nchunks, edge_split)
    f = pl.kernel(
        body,
        out_type=jax.ShapeDtypeStruct((2 * N, HH), jnp.float32),
        mesh=mesh,
        scratch_types=(
            [pltpu.VMEM((K,), jnp.int32)] * 8
            + [pltpu.VMEM((K, HH), jnp.float32)] * 3
            + [pltpu.VMEM_SHARED((N, HH), jnp.float32)]
            + [pltpu.SemaphoreType.DMA] * 10
        ),
    )
    return f(table, srcs, dsts)


R = 1000
NG = N // R


def _full_spec(shape):
    return pl.BlockSpec(shape, lambda i: (0,) * len(shape))


def _mlp1_body(x_ref, agg_ref, eps_ref, w1_ref, b1_ref, w2_ref, b2_ref,
               g_ref, bt_ref, out_ref):
    s = 1.0 / jnp.sqrt(1.0 + 1e-5)
    a = (1.0 + eps_ref[0, 0]) * x_ref[...] + agg_ref[0] + agg_ref[1]
    h = jnp.maximum(jnp.dot(a, w1_ref[...],
                            preferred_element_type=jnp.float32) + b1_ref[...], 0.0)
    h = jnp.maximum(jnp.dot(h, w2_ref[...],
                            preferred_element_type=jnp.float32) + b2_ref[...], 0.0)
    h = h * (s * g_ref[...]) + bt_ref[...]
    out_ref[0] = h[:, :HH]
    out_ref[1] = h[:, HH:]


def _mlp1(x, agg, eps, w1, b1, w2, b2, g, bt):
    return pl.pallas_call(
        _mlp1_body,
        grid=(NG,),
        in_specs=[
            pl.BlockSpec((R, F_IN), lambda i: (i, 0)),
            pl.BlockSpec((2, R, HH), lambda i: (0, i, 0)),
            pl.BlockSpec(memory_space=pltpu.SMEM),
            _full_spec((F_IN, H)),
            _full_spec((H,)),
            _full_spec((H, H)),
            _full_spec((H,)),
            _full_spec((H,)),
            _full_spec((H,)),
        ],
        out_specs=pl.BlockSpec((2, R, HH), lambda i: (0, i, 0)),
        out_shape=jax.ShapeDtypeStruct((2, N, HH), jnp.float32),
    )(x, agg, eps, w1, b1, w2, b2, g, bt)


def _mlp23_body(h_ref, agg_ref, eps_ref, w1_ref, b1_ref, w2_ref, b2_ref,
                g_ref, bt_ref, out_ref):
    s = 1.0 / jnp.sqrt(1.0 + 1e-5)
    hcat = jnp.concatenate([h_ref[0], h_ref[1]], axis=1)
    acat = jnp.concatenate([agg_ref[0], agg_ref[1]], axis=1)
    a = (1.0 + eps_ref[0, 0]) * hcat + acat
    h = jnp.maximum(jnp.dot(a, w1_ref[...],
                            preferred_element_type=jnp.float32) + b1_ref[...], 0.0)
    h = jnp.maximum(jnp.dot(h, w2_ref[...],
                            preferred_element_type=jnp.float32) + b2_ref[...], 0.0)
    h = h * (s * g_ref[...]) + bt_ref[...]
    out_ref[0] = h[:, :HH]
    out_ref[1] = h[:, HH:]


def _mlp2(h, agg, eps, w1, b1, w2, b2, g, bt):
    return pl.pallas_call(
        _mlp23_body,
        grid=(NG,),
        in_specs=[
            pl.BlockSpec((2, R, HH), lambda i: (0, i, 0)),
            pl.BlockSpec((2, R, HH), lambda i: (0, i, 0)),
            pl.BlockSpec(memory_space=pltpu.SMEM),
            _full_spec((H, H)),
            _full_spec((H,)),
            _full_spec((H, H)),
            _full_spec((H,)),
            _full_spec((H,)),
            _full_spec((H,)),
        ],
        out_specs=pl.BlockSpec((2, R, HH), lambda i: (0, i, 0)),
        out_shape=jax.ShapeDtypeStruct((2, N, HH), jnp.float32),
    )(h, agg, eps, w1, b1, w2, b2, g, bt)


def _mlp3_pool_head_body(h_ref, agg_ref, batch_ref, eps_ref, w1_ref, b1_ref,
                         w2_ref, b2_ref, g_ref, bt_ref, l1w_ref, l1b_ref,
                         l2w_ref, l2b_ref, out_ref, sums_ref, counts_ref):
    i = pl.program_id(0)

    @pl.when(i == 0)
    def _():
        sums_ref[...] = jnp.zeros_like(sums_ref)
        counts_ref[...] = jnp.zeros_like(counts_ref)

    s = 1.0 / jnp.sqrt(1.0 + 1e-5)
    hcat = jnp.concatenate([h_ref[0], h_ref[1]], axis=1)
    acat = jnp.concatenate([agg_ref[0], agg_ref[1]], axis=1)
    a = (1.0 + eps_ref[0, 0]) * hcat + acat
    h = jnp.maximum(jnp.dot(a, w1_ref[...],
                            preferred_element_type=jnp.float32) + b1_ref[...], 0.0)
    h = jnp.maximum(jnp.dot(h, w2_ref[...],
                            preferred_element_type=jnp.float32) + b2_ref[...], 0.0)
    h = h * (s * g_ref[...]) + bt_ref[...]

    b = batch_ref[0, 0, :]
    onehot = (b[:, None] == lax.broadcasted_iota(jnp.int32, (R, G), 1)
              ).astype(jnp.float32)
    sums_ref[...] += lax.dot_general(onehot, h, (((0,), (0,)), ((), ())),
                                     preferred_element_type=jnp.float32)
    counts_ref[...] += jnp.sum(onehot, axis=0)[None, :]

    @pl.when(i == NG - 1)
    def _():
        inv = 1.0 / jnp.maximum(counts_ref[0, :], 1.0)
        pooled = sums_ref[...] * inv[:, None]
        z = jnp.maximum(jnp.dot(pooled, l1w_ref[...],
                                preferred_element_type=jnp.float32) + l1b_ref[...],
                        0.0)
        z = jnp.dot(z, l2w_ref[...],
                    preferred_element_type=jnp.float32) + l2b_ref[...]
        m = jnp.max(z, axis=1, keepdims=True)
        lse = m + jnp.log(jnp.sum(jnp.exp(z - m), axis=1, keepdims=True))
        out_ref[...] = z - lse


def _mlp3_pool_head(h, agg, batch3, eps, w1, b1, w2, b2, g, bt,
                    l1w, l1b, l2w, l2b):
    return pl.pallas_call(
        _mlp3_pool_head_body,
        grid=(NG,),
        in_specs=[
            pl.BlockSpec((2, R, HH), lambda i: (0, i, 0)),
            pl.BlockSpec((2, R, HH), lambda i: (0, i, 0)),
            pl.BlockSpec((1, 1, R), lambda i: (i, 0, 0)),
            pl.BlockSpec(memory_space=pltpu.SMEM),
            _full_spec((H, H)),
            _full_spec((H,)),
            _full_spec((H, H)),
            _full_spec((H,)),
            _full_spec((H,)),
            _full_spec((H,)),
            _full_spec((H, H)),
            _full_spec((H,)),
            _full_spec((H, C)),
            _full_spec((C,)),
        ],
        out_specs=pl.BlockSpec((G, C), lambda i: (0, 0)),
        out_shape=jax.ShapeDtypeStruct((G, C), jnp.float32),
        scratch_shapes=[
            pltpu.VMEM((G, H), jnp.float32),
            pltpu.VMEM((1, G), jnp.float32),
        ],
    )(h, agg, batch3, eps, w1, b1, w2, b2, g, bt, l1w, l1b, l2w, l2b)


def kernel(x, edge_index, batch, eps1, c1_W1, c1_b1, c1_W2, c1_b2, c1_g, c1_bt,
           eps2, c2_W1, c2_b1, c2_W2, c2_b2, c2_g, c2_bt,
           eps3, c3_W1, c3_b1, c3_W2, c3_b2, c3_g, c3_bt,
           lin1_W, lin1_b, lin2_W, lin2_b):
    src = edge_index[0]
    dst = edge_index[1]
    batch3 = batch.reshape(NG, 1, R)
    e1 = jnp.reshape(eps1, (1, 1))
    e2 = jnp.reshape(eps2, (1, 1))
    e3 = jnp.reshape(eps3, (1, 1))
    agg1 = _sc_agg(x, src, dst, E // (2 * K), edge_split=True)
    h1 = _mlp1(x, agg1.reshape(2, N, HH), e1,
               c1_W1, c1_b1, c1_W2, c1_b2, c1_g, c1_bt)
    agg2 = _sc_agg(h1.reshape(2 * N, HH), src, dst, E // K, edge_split=False)
    h2 = _mlp2(h1, agg2.reshape(2, N, HH), e2,
               c2_W1, c2_b1, c2_W2, c2_b2, c2_g, c2_bt)
    agg3 = _sc_agg(h2.reshape(2 * N, HH), src, dst, E // K, edge_split=False)
    return _mlp3_pool_head(h2, agg3.reshape(2, N, HH), batch3,
                           e3, c3_W1, c3_b1, c3_W2, c3_b2,
                           c3_g, c3_bt, lin1_W, lin1_b, lin2_W, lin2_b)
